# baseline (speedup 1.0000x reference)
import functools

import jax
import jax.numpy as jnp
from jax.experimental import pallas as pl
from jax.experimental.pallas import tpu as pltpu

VOCAB = 100000
D_MODEL = 1024
TILE_V = 2048


def _gather_body(idx_ref, embed_ref, out_ref):
    out_ref[...] = embed_ref[...]


def _matmul_body(h_ref, w_ref, o_ref):
    o_ref[...] = jnp.dot(h_ref[...], w_ref[...],
                         preferred_element_type=jnp.float32)


@jax.jit
def kernel(x, embed, w_out):
    b, s = x.shape
    n_tok = b * s
    vocab = w_out.shape[1]
    d = embed.shape[1]
    idx = x.reshape(n_tok)

    h = pl.pallas_call(
        _gather_body,
        grid_spec=pltpu.PrefetchScalarGridSpec(
            num_scalar_prefetch=1,
            grid=(n_tok,),
            in_specs=[pl.BlockSpec((1, 1, d),
                                   lambda t, idx_ref: (idx_ref[t], 0, 0))],
            out_specs=pl.BlockSpec((1, 1, d), lambda t, idx_ref: (t, 0, 0)),
        ),
        out_shape=jax.ShapeDtypeStruct((n_tok, 1, d), jnp.float32),
    )(idx, embed.reshape(-1, 1, d))
    h = h.reshape(n_tok, d)

    n_v = pl.cdiv(vocab, TILE_V)
    logits = pl.pallas_call(
        _matmul_body,
        grid=(n_v,),
        in_specs=[
            pl.BlockSpec((n_tok, d), lambda v: (0, 0)),
            pl.BlockSpec((d, TILE_V), lambda v: (0, v)),
        ],
        out_specs=pl.BlockSpec((n_tok, TILE_V), lambda v: (0, v)),
        out_shape=jax.ShapeDtypeStruct((n_tok, vocab), jnp.float32),
        compiler_params=pltpu.CompilerParams(
            dimension_semantics=("arbitrary",),
        ),
    )(h, w_out)

    return logits.reshape(b, s, vocab)
